# Initial kernel scaffold; baseline (speedup 1.0000x reference)
import jax
import jax.numpy as jnp
from jax.experimental import pallas as pl


def kernel(x, edge_index, edge_weight, Wxz, bxz, Whz, bhz, Wxr, bxr, Whr, bhr, Wxh, bxh, Whh, bhh, Wlin, blin):
    raise NotImplementedError("write your pallas kernel here")


# speedup vs baseline: 13.8405x; 13.8405x over previous
import functools

import jax
import jax.numpy as jnp
from jax import lax
from jax.experimental import pallas as pl
from jax.experimental.pallas import tpu as pltpu
from jax.experimental.pallas import tpu_sc as plsc

N = 10000
E = 320000
NPAD = 10240
F = 128

NC = 2
NS = 16
NW = NC * NS
EPT = E // NW
BLK = 400
NBLK = EPT // BLK
CH = 16
CPB = BLK // CH

DROWS = NPAD // 16

_mesh = plsc.VectorSubcoreMesh(core_axis_name="c", subcore_axis_name="s")


@functools.partial(
    pl.kernel,
    mesh=_mesh,
    out_type=jax.ShapeDtypeStruct((NW, NPAD), jnp.float32),
    scratch_types=[
        pltpu.VMEM((NPAD,), jnp.float32),
        pltpu.VMEM((BLK,), jnp.int32),
        pltpu.VMEM((BLK,), jnp.int32),
        pltpu.VMEM((BLK,), jnp.float32),
    ],
    compiler_params=pltpu.CompilerParams(needs_layout_passes=False),
)
def _deg_call(src_hbm, dst_hbm, w_hbm, out_hbm, deg_v, src_v, dst_v, w_v):
    cid = lax.axis_index("c")
    sid = lax.axis_index("s")
    wid = cid * NS + sid

    def _zero_row(i, _):
        deg_v[pl.ds(i * 16, 16)] = jnp.zeros((16,), jnp.float32)
        return _
    lax.fori_loop(0, NPAD // 16, _zero_row, None)

    def _block(b, _):
        base = wid * EPT + b * BLK
        pltpu.sync_copy(src_hbm.at[pl.ds(base, BLK)], src_v)
        pltpu.sync_copy(dst_hbm.at[pl.ds(base, BLK)], dst_v)
        pltpu.sync_copy(w_hbm.at[pl.ds(base, BLK)], w_v)

        def _chunk(c, _):
            s16 = src_v[pl.ds(c * CH, CH)]
            d16 = dst_v[pl.ds(c * CH, CH)]
            w16 = w_v[pl.ds(c * CH, CH)]
            wm = jnp.where(s16 == d16, 0.0, w16)
            plsc.addupdate_scatter(deg_v, [s16], wm)
            return _
        lax.fori_loop(0, CPB, _chunk, None)
        return _
    lax.fori_loop(0, NBLK, _block, None)

    pltpu.sync_copy(deg_v, out_hbm.at[wid])


def _dis_body(degp_ref, dis_ref):
    d = jnp.sum(degp_ref[...], axis=0)
    dis_ref[...] = jnp.where(d > 0, lax.rsqrt(d), 0.0)


def _dis_call(degp):
    return pl.pallas_call(
        _dis_body,
        out_shape=jax.ShapeDtypeStruct((NPAD // 128, 128), jnp.float32),
    )(degp)


@functools.partial(
    pl.kernel,
    mesh=_mesh,
    out_type=jax.ShapeDtypeStruct((NC, NPAD, F), jnp.float32),
    scratch_types=[
        pltpu.VMEM((NPAD,), jnp.float32),
        pltpu.VMEM((BLK,), jnp.int32),
        pltpu.VMEM((BLK,), jnp.int32),
        pltpu.VMEM((BLK,), jnp.float32),
        pltpu.VMEM((CH, F), jnp.float32),
        pltpu.VMEM((40, F), jnp.float32),
        pltpu.VMEM_SHARED((NPAD, F), jnp.float32),
        pltpu.SemaphoreType.DMA,
    ],
    compiler_params=pltpu.CompilerParams(needs_layout_passes=False),
)
def _spmm_call(x_hbm, src_hbm, dst_hbm, w_hbm, dis_hbm, out_hbm,
               dis_v, src_v, dst_v, w_v, rows_v, zero_v, s_sh, sem):
    cid = lax.axis_index("c")
    sid = lax.axis_index("s")
    wid = cid * NS + sid

    pltpu.sync_copy(dis_hbm, dis_v)

    def _zero_row(i, _):
        for j in range(F // 16):
            zero_v[i, pl.ds(j * 16, 16)] = jnp.zeros((16,), jnp.float32)
        return _
    lax.fori_loop(0, 40, _zero_row, None)

    rows_per_tile = NPAD // NS

    def _zinit(k, _):
        pltpu.sync_copy(zero_v, s_sh.at[pl.ds(sid * rows_per_tile + k * 40, 40)])
        return _
    lax.fori_loop(0, rows_per_tile // 40, _zinit, None)
    plsc.subcore_barrier()

    def _block(b, _):
        base = wid * EPT + b * BLK
        pltpu.sync_copy(src_hbm.at[pl.ds(base, BLK)], src_v)
        pltpu.sync_copy(dst_hbm.at[pl.ds(base, BLK)], dst_v)
        pltpu.sync_copy(w_hbm.at[pl.ds(base, BLK)], w_v)

        def _chunk(c, _):
            s16 = src_v[pl.ds(c * CH, CH)]
            d16 = dst_v[pl.ds(c * CH, CH)]
            w16 = w_v[pl.ds(c * CH, CH)]
            wm = jnp.where(s16 == d16, 0.0, w16)
            diss = plsc.load_gather(dis_v, [s16])
            disd = plsc.load_gather(dis_v, [d16])
            nw16 = -(diss * wm * disd)
            pltpu.async_copy(x_hbm.at[s16], rows_v, sem).wait()
            for e in range(CH):
                sc = jnp.full((16,), nw16[e])
                for j in range(F // 16):
                    rows_v[e, pl.ds(j * 16, 16)] = (
                        rows_v[e, pl.ds(j * 16, 16)] * sc)
            pltpu.sync_copy(rows_v, s_sh.at[d16], add=True)
            return _
        lax.fori_loop(0, CPB, _chunk, None)
        return _
    lax.fori_loop(0, NBLK, _block, None)
    plsc.subcore_barrier()

    r = sid * rows_per_tile
    pltpu.sync_copy(s_sh.at[pl.ds(r, rows_per_tile)],
                    out_hbm.at[cid, pl.ds(r, rows_per_tile)])


ROWS_B = 512


def _dense_body(x_ref, sp_ref, wz_ref, wh_ref, bz_ref, bh_ref, wl_ref,
                bl_ref, out_ref):
    xb = x_ref[...]
    s = sp_ref[0] + sp_ref[1]
    z = jax.nn.sigmoid(
        jnp.dot(xb, wz_ref[0], preferred_element_type=jnp.float32)
        + jnp.dot(s, wz_ref[1], preferred_element_type=jnp.float32)
        + bz_ref[...])
    ht = jnp.tanh(
        jnp.dot(xb, wh_ref[0], preferred_element_type=jnp.float32)
        + jnp.dot(s, wh_ref[1], preferred_element_type=jnp.float32)
        + bh_ref[...])
    h = jnp.maximum((1.0 - z) * ht, 0.0)
    out_ref[...] = (jnp.dot(h, wl_ref[...], preferred_element_type=jnp.float32)
                    + bl_ref[...])


def _dense_call(x, sp, Wxz, Wxh, bz, bh, Wlin, blin):
    grid = NPAD // ROWS_B
    return pl.pallas_call(
        _dense_body,
        grid=(grid,),
        in_specs=[
            pl.BlockSpec((ROWS_B, F), lambda i: (i, 0)),
            pl.BlockSpec((NC, ROWS_B, F), lambda i: (0, i, 0)),
            pl.BlockSpec((2, F, F), lambda i: (0, 0, 0)),
            pl.BlockSpec((2, F, F), lambda i: (0, 0, 0)),
            pl.BlockSpec((1, F), lambda i: (0, 0)),
            pl.BlockSpec((1, F), lambda i: (0, 0)),
            pl.BlockSpec((F, 1), lambda i: (0, 0)),
            pl.BlockSpec((1, 1), lambda i: (0, 0)),
        ],
        out_specs=pl.BlockSpec((ROWS_B, 1), lambda i: (i, 0)),
        out_shape=jax.ShapeDtypeStruct((NPAD, 1), jnp.float32),
    )(x, sp, Wxz, Wxh, bz, bh, Wlin, blin)


def kernel(x, edge_index, edge_weight, Wxz, bxz, Whz, bhz, Wxr, bxr, Whr,
           bhr, Wxh, bxh, Whh, bhh, Wlin, blin):
    src = edge_index[0]
    dst = edge_index[1]
    degp = _deg_call(src, dst, edge_weight)
    dis = _dis_call(degp.reshape(NW, NPAD // 128, 128))
    sp = _spmm_call(x, src, dst, edge_weight, dis.reshape(NPAD))
    xpad = jnp.pad(x, ((0, NPAD - N), (0, 0)))
    out = _dense_call(xpad, sp, Wxz, Wxh,
                      (bxz + bhz).reshape(1, F), (bxh + bhh).reshape(1, F),
                      Wlin, blin.reshape(1, 1))
    return out[:N]


# speedup vs baseline: 28.7712x; 2.0788x over previous
import functools

import jax
import jax.numpy as jnp
from jax import lax
from jax.experimental import pallas as pl
from jax.experimental.pallas import tpu as pltpu
from jax.experimental.pallas import tpu_sc as plsc

N = 10000
E = 320000
NPAD = 10240
F = 128

NC = 2
NS = 16
NW = NC * NS
EPT = E // NW
BLK = 400
NBLK = EPT // BLK
CH = 16
CPB = BLK // CH

DROWS = NPAD // 16

_mesh = plsc.VectorSubcoreMesh(core_axis_name="c", subcore_axis_name="s")


@functools.partial(
    pl.kernel,
    mesh=_mesh,
    out_type=jax.ShapeDtypeStruct((NW, NPAD), jnp.float32),
    scratch_types=[
        pltpu.VMEM((NPAD,), jnp.float32),
        pltpu.VMEM((EPT,), jnp.int32),
        pltpu.VMEM((EPT,), jnp.int32),
        pltpu.VMEM((EPT,), jnp.float32),
    ],
    compiler_params=pltpu.CompilerParams(needs_layout_passes=False),
)
def _deg_call(src_hbm, dst_hbm, w_hbm, out_hbm, deg_v, src_v, dst_v, w_v):
    cid = lax.axis_index("c")
    sid = lax.axis_index("s")
    wid = cid * NS + sid
    base = wid * EPT
    pltpu.sync_copy(src_hbm.at[pl.ds(base, EPT)], src_v)
    pltpu.sync_copy(dst_hbm.at[pl.ds(base, EPT)], dst_v)
    pltpu.sync_copy(w_hbm.at[pl.ds(base, EPT)], w_v)

    def _zero_row(i, _):
        deg_v[pl.ds(i * 16, 16)] = jnp.zeros((16,), jnp.float32)
        return _
    lax.fori_loop(0, NPAD // 16, _zero_row, None)

    def _chunk(c, _):
        s16 = src_v[pl.ds(c * CH, CH)]
        d16 = dst_v[pl.ds(c * CH, CH)]
        w16 = w_v[pl.ds(c * CH, CH)]
        wm = jnp.where(s16 == d16, 0.0, w16)
        plsc.addupdate_scatter(deg_v, [s16], wm)
        return _
    lax.fori_loop(0, EPT // CH, _chunk, None)

    pltpu.sync_copy(deg_v, out_hbm.at[wid])


def _dis_body(degp_ref, dis_ref):
    d = jnp.sum(degp_ref[...], axis=0)
    dis_ref[...] = jnp.where(d > 0, lax.rsqrt(d), 0.0)


def _dis_call(degp):
    return pl.pallas_call(
        _dis_body,
        out_shape=jax.ShapeDtypeStruct((NPAD // 128, 128), jnp.float32),
    )(degp)


GRP = 2000
NG = EPT // GRP
CPG = GRP // CH


@functools.partial(
    pl.kernel,
    mesh=_mesh,
    out_type=jax.ShapeDtypeStruct((NC, NPAD, F), jnp.float32),
    scratch_types=[
        pltpu.VMEM((NPAD,), jnp.float32),
        pltpu.VMEM((GRP,), jnp.int32),
        pltpu.VMEM((GRP,), jnp.int32),
        pltpu.VMEM((GRP,), jnp.float32),
        pltpu.VMEM((CH, F), jnp.float32),
        pltpu.VMEM((CH, F), jnp.float32),
        pltpu.VMEM((CH, F), jnp.float32),
        pltpu.VMEM((CH, F), jnp.float32),
        pltpu.VMEM((40, F), jnp.float32),
        pltpu.VMEM_SHARED((NPAD, F), jnp.float32),
        pltpu.SemaphoreType.DMA,
        pltpu.SemaphoreType.DMA,
        pltpu.SemaphoreType.DMA,
        pltpu.SemaphoreType.DMA,
    ],
    compiler_params=pltpu.CompilerParams(needs_layout_passes=False),
)
def _spmm_call(x_hbm, src_hbm, dst_hbm, w_hbm, dis_hbm, out_hbm,
               dis_v, src_v, dst_v, w_v, g0, g1, s0, s1, zero_v, s_sh,
               gsem0, gsem1, ssem0, ssem1):
    cid = lax.axis_index("c")
    sid = lax.axis_index("s")
    wid = cid * NS + sid

    pltpu.sync_copy(dis_hbm, dis_v)

    def _zero_row(i, _):
        for j in range(F // 16):
            zero_v[i, pl.ds(j * 16, 16)] = jnp.zeros((16,), jnp.float32)
        return _
    lax.fori_loop(0, 40, _zero_row, None)

    rows_per_tile = NPAD // NS

    def _zinit(k, _):
        pltpu.sync_copy(zero_v, s_sh.at[pl.ds(sid * rows_per_tile + k * 40, 40)])
        return _
    lax.fori_loop(0, rows_per_tile // 40, _zinit, None)
    plsc.subcore_barrier()

    gbuf = (g0, g1)
    sbuf = (s0, s1)
    gsems = (gsem0, gsem1)
    ssems = (ssem0, ssem1)

    def _process(c, b, first):
        gb, sb, gsem, ssem = gbuf[b], sbuf[b], gsems[b], ssems[b]
        s16 = src_v[pl.ds(c * CH, CH)]
        d16 = dst_v[pl.ds(c * CH, CH)]
        w16 = w_v[pl.ds(c * CH, CH)]
        wm = jnp.where(s16 == d16, 0.0, w16)
        nw16 = -(plsc.load_gather(dis_v, [s16]) * wm
                 * plsc.load_gather(dis_v, [d16]))
        pltpu.make_async_copy(x_hbm.at[s16], gb, gsem).wait()
        if not first:
            pltpu.make_async_copy(x_hbm.at[pl.ds(0, CH)], sb, ssem).wait()
        for e in range(CH):
            sc = jnp.full((16,), nw16[e])
            for j in range(F // 16):
                sb[e, pl.ds(j * 16, 16)] = gb[e, pl.ds(j * 16, 16)] * sc
        c2 = c + 2

        @pl.when(c2 < CPG)
        def _():
            pltpu.async_copy(x_hbm.at[src_v[pl.ds(c2 * CH, CH)]], gb, gsem)
        pltpu.async_copy(sb, s_sh.at[d16], ssem, add=True)

    def _group(g, first):
        base = wid * EPT + g * GRP
        pltpu.sync_copy(src_hbm.at[pl.ds(base, GRP)], src_v)
        pltpu.sync_copy(dst_hbm.at[pl.ds(base, GRP)], dst_v)
        pltpu.sync_copy(w_hbm.at[pl.ds(base, GRP)], w_v)
        pltpu.async_copy(x_hbm.at[src_v[pl.ds(0, CH)]], g0, gsem0)
        pltpu.async_copy(x_hbm.at[src_v[pl.ds(CH, CH)]], g1, gsem1)
        _process(jnp.int32(0), 0, first)
        _process(jnp.int32(1), 1, first)

        def _pair(t, _):
            _process(2 * t, 0, False)
            _process(2 * t + 1, 1, False)
            return _
        lax.fori_loop(1, CPG // 2, _pair, None)
        _process(jnp.int32(CPG - 1), 0, False)

    _group(jnp.int32(0), True)

    def _grp_body(g, _):
        _group(g, False)
        return _
    lax.fori_loop(1, NG, _grp_body, None)

    pltpu.make_async_copy(x_hbm.at[pl.ds(0, CH)], s0, ssem0).wait()
    pltpu.make_async_copy(x_hbm.at[pl.ds(0, CH)], s1, ssem1).wait()
    plsc.subcore_barrier()

    r = sid * rows_per_tile
    pltpu.sync_copy(s_sh.at[pl.ds(r, rows_per_tile)],
                    out_hbm.at[cid, pl.ds(r, rows_per_tile)])


ROWS_B = 512


def _dense_body(x_ref, sp_ref, wz_ref, wh_ref, bz_ref, bh_ref, wl_ref,
                bl_ref, out_ref):
    xb = x_ref[...]
    s = sp_ref[0] + sp_ref[1]
    z = jax.nn.sigmoid(
        jnp.dot(xb, wz_ref[0], preferred_element_type=jnp.float32)
        + jnp.dot(s, wz_ref[1], preferred_element_type=jnp.float32)
        + bz_ref[...])
    ht = jnp.tanh(
        jnp.dot(xb, wh_ref[0], preferred_element_type=jnp.float32)
        + jnp.dot(s, wh_ref[1], preferred_element_type=jnp.float32)
        + bh_ref[...])
    h = jnp.maximum((1.0 - z) * ht, 0.0)
    out_ref[...] = (jnp.dot(h, wl_ref[...], preferred_element_type=jnp.float32)
                    + bl_ref[...])


def _dense_call(x, sp, Wxz, Wxh, bz, bh, Wlin, blin):
    grid = NPAD // ROWS_B
    return pl.pallas_call(
        _dense_body,
        grid=(grid,),
        in_specs=[
            pl.BlockSpec((ROWS_B, F), lambda i: (i, 0)),
            pl.BlockSpec((NC, ROWS_B, F), lambda i: (0, i, 0)),
            pl.BlockSpec((2, F, F), lambda i: (0, 0, 0)),
            pl.BlockSpec((2, F, F), lambda i: (0, 0, 0)),
            pl.BlockSpec((1, F), lambda i: (0, 0)),
            pl.BlockSpec((1, F), lambda i: (0, 0)),
            pl.BlockSpec((F, 1), lambda i: (0, 0)),
            pl.BlockSpec((1, 1), lambda i: (0, 0)),
        ],
        out_specs=pl.BlockSpec((ROWS_B, 1), lambda i: (i, 0)),
        out_shape=jax.ShapeDtypeStruct((NPAD, 1), jnp.float32),
    )(x, sp, Wxz, Wxh, bz, bh, Wlin, blin)


def kernel(x, edge_index, edge_weight, Wxz, bxz, Whz, bhz, Wxr, bxr, Whr,
           bhr, Wxh, bxh, Whh, bhh, Wlin, blin):
    src = edge_index[0]
    dst = edge_index[1]
    degp = _deg_call(src, dst, edge_weight)
    dis = _dis_call(degp.reshape(NW, NPAD // 128, 128))
    sp = _spmm_call(x, src, dst, edge_weight, dis.reshape(NPAD))
    xpad = jnp.pad(x, ((0, NPAD - N), (0, 0)))
    out = _dense_call(xpad, sp, Wxz, Wxh,
                      (bxz + bhz).reshape(1, F), (bxh + bhh).reshape(1, F),
                      Wlin, blin.reshape(1, 1))
    return out[:N]


# speedup vs baseline: 41.9527x; 1.4581x over previous
import functools

import jax
import jax.numpy as jnp
from jax import lax
from jax.experimental import pallas as pl
from jax.experimental.pallas import tpu as pltpu
from jax.experimental.pallas import tpu_sc as plsc

N = 10000
E = 320000
NPAD = 10240
F = 128

NC = 2
NS = 16
NW = NC * NS
EPT = E // NW
BLK = 400
NBLK = EPT // BLK
CH = 16
CPB = BLK // CH

DROWS = NPAD // 16

_mesh = plsc.VectorSubcoreMesh(core_axis_name="c", subcore_axis_name="s")


@functools.partial(
    pl.kernel,
    mesh=_mesh,
    out_type=jax.ShapeDtypeStruct((NW, NPAD), jnp.float32),
    scratch_types=[
        pltpu.VMEM((NPAD,), jnp.float32),
        pltpu.VMEM((EPT,), jnp.int32),
        pltpu.VMEM((EPT,), jnp.int32),
        pltpu.VMEM((EPT,), jnp.float32),
    ],
    compiler_params=pltpu.CompilerParams(needs_layout_passes=False),
)
def _deg_call(src_hbm, dst_hbm, w_hbm, out_hbm, deg_v, src_v, dst_v, w_v):
    cid = lax.axis_index("c")
    sid = lax.axis_index("s")
    wid = cid * NS + sid
    base = wid * EPT
    pltpu.sync_copy(src_hbm.at[pl.ds(base, EPT)], src_v)
    pltpu.sync_copy(dst_hbm.at[pl.ds(base, EPT)], dst_v)
    pltpu.sync_copy(w_hbm.at[pl.ds(base, EPT)], w_v)

    def _zero_row(i, _):
        deg_v[pl.ds(i * 16, 16)] = jnp.zeros((16,), jnp.float32)
        return _
    lax.fori_loop(0, NPAD // 16, _zero_row, None)

    def _chunk(c, _):
        s16 = src_v[pl.ds(c * CH, CH)]
        d16 = dst_v[pl.ds(c * CH, CH)]
        w16 = w_v[pl.ds(c * CH, CH)]
        wm = jnp.where(s16 == d16, 0.0, w16)
        plsc.addupdate_scatter(deg_v, [s16], wm)
        return _
    lax.fori_loop(0, EPT // CH, _chunk, None)

    pltpu.sync_copy(deg_v, out_hbm.at[wid])


def _dis_body(degp_ref, dis_ref):
    d = jnp.sum(degp_ref[...], axis=0)
    dis_ref[...] = jnp.where(d > 0, lax.rsqrt(d), 0.0)


def _dis_call(degp):
    return pl.pallas_call(
        _dis_body,
        out_shape=jax.ShapeDtypeStruct((NPAD // 128, 128), jnp.float32),
    )(degp)


GRP = 2000
NG = EPT // GRP
CPG = GRP // CH


@functools.partial(
    pl.kernel,
    mesh=_mesh,
    out_type=jax.ShapeDtypeStruct((NC, NPAD, F), jnp.float32),
    scratch_types=[
        pltpu.VMEM((NPAD,), jnp.float32),
        pltpu.VMEM((GRP,), jnp.int32),
        pltpu.VMEM((GRP,), jnp.int32),
        pltpu.VMEM((GRP,), jnp.float32),
        pltpu.VMEM((CH, F), jnp.float32),
        pltpu.VMEM((CH, F), jnp.float32),
        pltpu.VMEM((CH, F), jnp.float32),
        pltpu.VMEM((CH, F), jnp.float32),
        pltpu.VMEM((CH, F), jnp.float32),
        pltpu.VMEM((CH, F), jnp.float32),
        pltpu.VMEM((CH, F), jnp.float32),
        pltpu.VMEM((CH, F), jnp.float32),
        pltpu.VMEM((40, F), jnp.float32),
        pltpu.VMEM_SHARED((NPAD, F), jnp.float32),
        pltpu.SemaphoreType.DMA,
        pltpu.SemaphoreType.DMA,
        pltpu.SemaphoreType.DMA,
        pltpu.SemaphoreType.DMA,
        pltpu.SemaphoreType.DMA,
        pltpu.SemaphoreType.DMA,
        pltpu.SemaphoreType.DMA,
        pltpu.SemaphoreType.DMA,
    ],
    compiler_params=pltpu.CompilerParams(needs_layout_passes=False),
)
def _spmm_call(x_hbm, src_hbm, dst_hbm, w_hbm, dis_hbm, out_hbm,
               dis_v, src_v, dst_v, w_v, g0, g1, g2, g3, s0, s1, s2, s3,
               zero_v, s_sh,
               gsem0, gsem1, gsem2, gsem3, ssem0, ssem1, ssem2, ssem3):
    cid = lax.axis_index("c")
    sid = lax.axis_index("s")
    wid = cid * NS + sid

    pltpu.sync_copy(dis_hbm, dis_v)

    def _zero_row(i, _):
        for j in range(F // 16):
            zero_v[i, pl.ds(j * 16, 16)] = jnp.zeros((16,), jnp.float32)
        return _
    lax.fori_loop(0, 40, _zero_row, None)

    rows_per_tile = NPAD // NS

    def _zinit(k, _):
        pltpu.sync_copy(zero_v, s_sh.at[pl.ds(sid * rows_per_tile + k * 40, 40)])
        return _
    lax.fori_loop(0, rows_per_tile // 40, _zinit, None)
    plsc.subcore_barrier()

    gbuf = (g0, g1, g2, g3)
    sbuf = (s0, s1, s2, s3)
    gsems = (gsem0, gsem1, gsem2, gsem3)
    ssems = (ssem0, ssem1, ssem2, ssem3)
    NBUF = 4

    def _process(c, b, first):
        gb, sb, gsem, ssem = gbuf[b], sbuf[b], gsems[b], ssems[b]
        s16 = src_v[pl.ds(c * CH, CH)]
        d16 = dst_v[pl.ds(c * CH, CH)]
        w16 = w_v[pl.ds(c * CH, CH)]
        wm = jnp.where(s16 == d16, 0.0, w16)
        nw16 = -(plsc.load_gather(dis_v, [s16]) * wm
                 * plsc.load_gather(dis_v, [d16]))
        pltpu.make_async_copy(x_hbm.at[s16], gb, gsem).wait()
        if not first:
            pltpu.make_async_copy(x_hbm.at[pl.ds(0, CH)], sb, ssem).wait()
        for e in range(CH):
            sc = jnp.full((16,), nw16[e])
            for j in range(F // 16):
                sb[e, pl.ds(j * 16, 16)] = gb[e, pl.ds(j * 16, 16)] * sc
        c2 = c + NBUF

        @pl.when(c2 < CPG)
        def _():
            pltpu.async_copy(x_hbm.at[src_v[pl.ds(c2 * CH, CH)]], gb, gsem)
        pltpu.async_copy(sb, s_sh.at[d16], ssem, add=True)

    def _group(g, first):
        base = wid * EPT + g * GRP
        pltpu.sync_copy(src_hbm.at[pl.ds(base, GRP)], src_v)
        pltpu.sync_copy(dst_hbm.at[pl.ds(base, GRP)], dst_v)
        pltpu.sync_copy(w_hbm.at[pl.ds(base, GRP)], w_v)
        for b in range(NBUF):
            pltpu.async_copy(x_hbm.at[src_v[pl.ds(b * CH, CH)]],
                             gbuf[b], gsems[b])
        for b in range(NBUF):
            _process(jnp.int32(b), b, first)

        def _quad(t, _):
            for b in range(NBUF):
                _process(NBUF * t + b, b, False)
            return _
        lax.fori_loop(1, CPG // NBUF, _quad, None)
        _process(jnp.int32(CPG - 1), 0, False)

    _group(jnp.int32(0), True)

    def _grp_body(g, _):
        _group(g, False)
        return _
    lax.fori_loop(1, NG, _grp_body, None)

    for b in range(NBUF):
        pltpu.make_async_copy(x_hbm.at[pl.ds(0, CH)], sbuf[b], ssems[b]).wait()
    plsc.subcore_barrier()

    r = sid * rows_per_tile
    pltpu.sync_copy(s_sh.at[pl.ds(r, rows_per_tile)],
                    out_hbm.at[cid, pl.ds(r, rows_per_tile)])


ROWS_B = 512


def _dense_body(x_ref, sp_ref, wz_ref, wh_ref, bz_ref, bh_ref, wl_ref,
                bl_ref, out_ref):
    xb = x_ref[...]
    s = sp_ref[0] + sp_ref[1]
    z = jax.nn.sigmoid(
        jnp.dot(xb, wz_ref[0], preferred_element_type=jnp.float32)
        + jnp.dot(s, wz_ref[1], preferred_element_type=jnp.float32)
        + bz_ref[...])
    ht = jnp.tanh(
        jnp.dot(xb, wh_ref[0], preferred_element_type=jnp.float32)
        + jnp.dot(s, wh_ref[1], preferred_element_type=jnp.float32)
        + bh_ref[...])
    h = jnp.maximum((1.0 - z) * ht, 0.0)
    out_ref[...] = (jnp.dot(h, wl_ref[...], preferred_element_type=jnp.float32)
                    + bl_ref[...])


def _dense_call(x, sp, Wxz, Wxh, bz, bh, Wlin, blin):
    grid = NPAD // ROWS_B
    return pl.pallas_call(
        _dense_body,
        grid=(grid,),
        in_specs=[
            pl.BlockSpec((ROWS_B, F), lambda i: (i, 0)),
            pl.BlockSpec((NC, ROWS_B, F), lambda i: (0, i, 0)),
            pl.BlockSpec((2, F, F), lambda i: (0, 0, 0)),
            pl.BlockSpec((2, F, F), lambda i: (0, 0, 0)),
            pl.BlockSpec((1, F), lambda i: (0, 0)),
            pl.BlockSpec((1, F), lambda i: (0, 0)),
            pl.BlockSpec((F, 1), lambda i: (0, 0)),
            pl.BlockSpec((1, 1), lambda i: (0, 0)),
        ],
        out_specs=pl.BlockSpec((ROWS_B, 1), lambda i: (i, 0)),
        out_shape=jax.ShapeDtypeStruct((NPAD, 1), jnp.float32),
    )(x, sp, Wxz, Wxh, bz, bh, Wlin, blin)


def kernel(x, edge_index, edge_weight, Wxz, bxz, Whz, bhz, Wxr, bxr, Whr,
           bhr, Wxh, bxh, Whh, bhh, Wlin, blin):
    src = edge_index[0]
    dst = edge_index[1]
    degp = _deg_call(src, dst, edge_weight)
    dis = _dis_call(degp.reshape(NW, NPAD // 128, 128))
    sp = _spmm_call(x, src, dst, edge_weight, dis.reshape(NPAD))
    xpad = jnp.pad(x, ((0, NPAD - N), (0, 0)))
    out = _dense_call(xpad, sp, Wxz, Wxh,
                      (bxz + bhz).reshape(1, F), (bxh + bhh).reshape(1, F),
                      Wlin, blin.reshape(1, 1))
    return out[:N]


# speedup vs baseline: 45.6610x; 1.0884x over previous
import functools

import jax
import jax.numpy as jnp
from jax import lax
from jax.experimental import pallas as pl
from jax.experimental.pallas import tpu as pltpu
from jax.experimental.pallas import tpu_sc as plsc

N = 10000
E = 320000
NPAD = 10240
F = 128

NC = 2
NS = 16
NW = NC * NS
EPT = E // NW
BLK = 400
NBLK = EPT // BLK
CH = 16
CPB = BLK // CH

DROWS = NPAD // 16

_mesh = plsc.VectorSubcoreMesh(core_axis_name="c", subcore_axis_name="s")


@functools.partial(
    pl.kernel,
    mesh=_mesh,
    out_type=jax.ShapeDtypeStruct((NW, NPAD), jnp.float32),
    scratch_types=[
        pltpu.VMEM((NPAD,), jnp.float32),
        pltpu.VMEM((EPT,), jnp.int32),
        pltpu.VMEM((EPT,), jnp.int32),
        pltpu.VMEM((EPT,), jnp.float32),
    ],
    compiler_params=pltpu.CompilerParams(needs_layout_passes=False),
)
def _deg_call(src_hbm, dst_hbm, w_hbm, out_hbm, deg_v, src_v, dst_v, w_v):
    cid = lax.axis_index("c")
    sid = lax.axis_index("s")
    wid = cid * NS + sid
    base = wid * EPT
    pltpu.sync_copy(src_hbm.at[pl.ds(base, EPT)], src_v)
    pltpu.sync_copy(dst_hbm.at[pl.ds(base, EPT)], dst_v)
    pltpu.sync_copy(w_hbm.at[pl.ds(base, EPT)], w_v)

    def _zero_row(i, _):
        deg_v[pl.ds(i * 16, 16)] = jnp.zeros((16,), jnp.float32)
        return _
    lax.fori_loop(0, NPAD // 16, _zero_row, None)

    def _chunk(c, _):
        s16 = src_v[pl.ds(c * CH, CH)]
        d16 = dst_v[pl.ds(c * CH, CH)]
        w16 = w_v[pl.ds(c * CH, CH)]
        wm = jnp.where(s16 == d16, 0.0, w16)
        plsc.addupdate_scatter(deg_v, [s16], wm)
        return _
    lax.fori_loop(0, EPT // CH, _chunk, None)

    pltpu.sync_copy(deg_v, out_hbm.at[wid])


def _dis_body(degp_ref, dis_ref):
    d = jnp.sum(degp_ref[...], axis=0)
    dis_ref[...] = jnp.where(d > 0, lax.rsqrt(d), 0.0)


def _dis_call(degp):
    return pl.pallas_call(
        _dis_body,
        out_shape=jax.ShapeDtypeStruct((NPAD // 128, 128), jnp.float32),
    )(degp)


GRP = 2000
NG = EPT // GRP
CPG = GRP // CH


@functools.partial(
    pl.kernel,
    mesh=_mesh,
    out_type=jax.ShapeDtypeStruct((NC, NPAD, F), jnp.float32),
    scratch_types=[
        pltpu.VMEM((NPAD,), jnp.float32),
        pltpu.VMEM((GRP,), jnp.int32),
        pltpu.VMEM((GRP,), jnp.int32),
        pltpu.VMEM((GRP,), jnp.float32),
        pltpu.VMEM((CH, F), jnp.float32),
        pltpu.VMEM((CH, F), jnp.float32),
        pltpu.VMEM((CH, F), jnp.float32),
        pltpu.VMEM((CH, F), jnp.float32),
        pltpu.VMEM((CH, F), jnp.float32),
        pltpu.VMEM((CH, F), jnp.float32),
        pltpu.VMEM((CH, F), jnp.float32),
        pltpu.VMEM((CH, F), jnp.float32),
        pltpu.VMEM((CH, F), jnp.float32),
        pltpu.VMEM((CH, F), jnp.float32),
        pltpu.VMEM((40, F), jnp.float32),
        pltpu.VMEM_SHARED((NPAD, F), jnp.float32),
        pltpu.SemaphoreType.DMA,
        pltpu.SemaphoreType.DMA,
        pltpu.SemaphoreType.DMA,
        pltpu.SemaphoreType.DMA,
        pltpu.SemaphoreType.DMA,
        pltpu.SemaphoreType.DMA,
        pltpu.SemaphoreType.DMA,
        pltpu.SemaphoreType.DMA,
        pltpu.SemaphoreType.DMA,
        pltpu.SemaphoreType.DMA,
    ],
    compiler_params=pltpu.CompilerParams(needs_layout_passes=False),
)
def _spmm_call(x_hbm, src_hbm, dst_hbm, w_hbm, dis_hbm, out_hbm,
               dis_v, src_v, dst_v, w_v, g0, g1, g2, g3, g4,
               s0, s1, s2, s3, s4, zero_v, s_sh,
               gsem0, gsem1, gsem2, gsem3, gsem4,
               ssem0, ssem1, ssem2, ssem3, ssem4):
    cid = lax.axis_index("c")
    sid = lax.axis_index("s")
    wid = cid * NS + sid

    pltpu.sync_copy(dis_hbm, dis_v)

    def _zero_row(i, _):
        for j in range(F // 16):
            zero_v[i, pl.ds(j * 16, 16)] = jnp.zeros((16,), jnp.float32)
        return _
    lax.fori_loop(0, 40, _zero_row, None)

    rows_per_tile = NPAD // NS

    def _zinit(k, _):
        pltpu.sync_copy(zero_v, s_sh.at[pl.ds(sid * rows_per_tile + k * 40, 40)])
        return _
    lax.fori_loop(0, rows_per_tile // 40, _zinit, None)
    plsc.subcore_barrier()

    gbuf = (g0, g1, g2, g3, g4)
    sbuf = (s0, s1, s2, s3, s4)
    gsems = (gsem0, gsem1, gsem2, gsem3, gsem4)
    ssems = (ssem0, ssem1, ssem2, ssem3, ssem4)
    NBUF = 5

    def _process(c, b, first):
        gb, sb, gsem, ssem = gbuf[b], sbuf[b], gsems[b], ssems[b]
        s16 = src_v[pl.ds(c * CH, CH)]
        d16 = dst_v[pl.ds(c * CH, CH)]
        w16 = w_v[pl.ds(c * CH, CH)]
        wm = jnp.where(s16 == d16, 0.0, w16)
        nw16 = -(plsc.load_gather(dis_v, [s16]) * wm
                 * plsc.load_gather(dis_v, [d16]))
        pltpu.make_async_copy(x_hbm.at[s16], gb, gsem).wait()
        if not first:
            pltpu.make_async_copy(x_hbm.at[pl.ds(0, CH)], sb, ssem).wait()
        for e in range(CH):
            sc = jnp.full((16,), nw16[e])
            for j in range(F // 16):
                sb[e, pl.ds(j * 16, 16)] = gb[e, pl.ds(j * 16, 16)] * sc
        c2 = c + NBUF

        @pl.when(c2 < CPG)
        def _():
            pltpu.async_copy(x_hbm.at[src_v[pl.ds(c2 * CH, CH)]], gb, gsem)
        pltpu.async_copy(sb, s_sh.at[d16], ssem, add=True)

    def _group(g, first):
        base = wid * EPT + g * GRP
        pltpu.sync_copy(src_hbm.at[pl.ds(base, GRP)], src_v)
        pltpu.sync_copy(dst_hbm.at[pl.ds(base, GRP)], dst_v)
        pltpu.sync_copy(w_hbm.at[pl.ds(base, GRP)], w_v)
        for b in range(NBUF):
            pltpu.async_copy(x_hbm.at[src_v[pl.ds(b * CH, CH)]],
                             gbuf[b], gsems[b])
        for b in range(NBUF):
            _process(jnp.int32(b), b, first)

        def _round(t, _):
            for b in range(NBUF):
                _process(NBUF * t + b, b, False)
            return _
        lax.fori_loop(1, CPG // NBUF, _round, None)

    _group(jnp.int32(0), True)

    def _grp_body(g, _):
        _group(g, False)
        return _
    lax.fori_loop(1, NG, _grp_body, None)

    for b in range(NBUF):
        pltpu.make_async_copy(x_hbm.at[pl.ds(0, CH)], sbuf[b], ssems[b]).wait()
    plsc.subcore_barrier()

    r = sid * rows_per_tile
    pltpu.sync_copy(s_sh.at[pl.ds(r, rows_per_tile)],
                    out_hbm.at[cid, pl.ds(r, rows_per_tile)])


ROWS_B = 512


def _dense_body(x_ref, sp_ref, wz_ref, wh_ref, bz_ref, bh_ref, wl_ref,
                bl_ref, out_ref):
    xb = x_ref[...]
    s = sp_ref[0] + sp_ref[1]
    z = jax.nn.sigmoid(
        jnp.dot(xb, wz_ref[0], preferred_element_type=jnp.float32)
        + jnp.dot(s, wz_ref[1], preferred_element_type=jnp.float32)
        + bz_ref[...])
    ht = jnp.tanh(
        jnp.dot(xb, wh_ref[0], preferred_element_type=jnp.float32)
        + jnp.dot(s, wh_ref[1], preferred_element_type=jnp.float32)
        + bh_ref[...])
    h = jnp.maximum((1.0 - z) * ht, 0.0)
    out_ref[...] = (jnp.dot(h, wl_ref[...], preferred_element_type=jnp.float32)
                    + bl_ref[...])


def _dense_call(x, sp, Wxz, Wxh, bz, bh, Wlin, blin):
    grid = NPAD // ROWS_B
    return pl.pallas_call(
        _dense_body,
        grid=(grid,),
        in_specs=[
            pl.BlockSpec((ROWS_B, F), lambda i: (i, 0)),
            pl.BlockSpec((NC, ROWS_B, F), lambda i: (0, i, 0)),
            pl.BlockSpec((2, F, F), lambda i: (0, 0, 0)),
            pl.BlockSpec((2, F, F), lambda i: (0, 0, 0)),
            pl.BlockSpec((1, F), lambda i: (0, 0)),
            pl.BlockSpec((1, F), lambda i: (0, 0)),
            pl.BlockSpec((F, 1), lambda i: (0, 0)),
            pl.BlockSpec((1, 1), lambda i: (0, 0)),
        ],
        out_specs=pl.BlockSpec((ROWS_B, 1), lambda i: (i, 0)),
        out_shape=jax.ShapeDtypeStruct((NPAD, 1), jnp.float32),
    )(x, sp, Wxz, Wxh, bz, bh, Wlin, blin)


def kernel(x, edge_index, edge_weight, Wxz, bxz, Whz, bhz, Wxr, bxr, Whr,
           bhr, Wxh, bxh, Whh, bhh, Wlin, blin):
    src = edge_index[0]
    dst = edge_index[1]
    degp = _deg_call(src, dst, edge_weight)
    dis = _dis_call(degp.reshape(NW, NPAD // 128, 128))
    sp = _spmm_call(x, src, dst, edge_weight, dis.reshape(NPAD))
    xpad = jnp.pad(x, ((0, NPAD - N), (0, 0)))
    out = _dense_call(xpad, sp, Wxz, Wxh,
                      (bxz + bhz).reshape(1, F), (bxh + bhh).reshape(1, F),
                      Wlin, blin.reshape(1, 1))
    return out[:N]
